# baseline (speedup 1.0000x reference)
import jax
import jax.numpy as jnp
from jax import lax
from jax.experimental import pallas as pl
from jax.experimental.pallas import tpu as pltpu
from jax.experimental.pallas import tpu_sc as plsc

E = 8
D = 1024
H = 4096
S = 2048
K = 512
EK = E * K
L = 16
NC = 2
NS = 16
NW = NC * NS
DH = D // NC

_PREC_GATE = lax.Precision.DEFAULT
_PREC_CP = lax.Precision.DEFAULT
_PREC_MLP = lax.Precision.HIGHEST


def _router_body(x_ref, gw_ref, w1_ref, b1_ref, w2_ref, b2_ref,
                 scores_ref, cap_ref):
    xb = x_ref[...]
    lg = lax.dot_general(gw_ref[...], xb, (((1,), (1,)), ((), ())),
                         preferred_element_type=jnp.float32,
                         precision=_PREC_GATE)
    m = jnp.max(lg, axis=0, keepdims=True)
    p = jnp.exp(lg - m)
    scores_ref[...] = p / jnp.sum(p, axis=0, keepdims=True)
    h = lax.dot_general(xb, w1_ref[...], (((1,), (1,)), ((), ())),
                        preferred_element_type=jnp.float32,
                        precision=_PREC_CP) + b1_ref[...]
    h = h * (1.0 / (1.0 + jnp.exp(-h)))
    cap_ref[...] = lax.dot_general(h, w2_ref[...], (((1,), (1,)), ((), ())),
                                   precision=_PREC_CP,
                                   preferred_element_type=jnp.float32) + b2_ref[...]


def _tc_router(xf, gate_weight, cp_w1, cp_b1, cp_w2, cp_b2):
    BS = 512
    grid = (S // BS,)
    return pl.pallas_call(
        _router_body,
        grid=grid,
        in_specs=[
            pl.BlockSpec((BS, D), lambda i: (i, 0)),
            pl.BlockSpec((E, D), lambda i: (0, 0)),
            pl.BlockSpec((D, D), lambda i: (0, 0)),
            pl.BlockSpec((D,), lambda i: (0,)),
            pl.BlockSpec((E, D), lambda i: (0, 0)),
            pl.BlockSpec((E,), lambda i: (0,)),
        ],
        out_specs=[
            pl.BlockSpec((E, BS), lambda i: (0, i)),
            pl.BlockSpec((BS, E), lambda i: (i, 0)),
        ],
        out_shape=[
            jax.ShapeDtypeStruct((E, S), jnp.float32),
            jax.ShapeDtypeStruct((S, E), jnp.float32),
        ],
    )(xf, gate_weight, cp_w1, cp_b1, cp_w2, cp_b2)


def _topk_body(scores_hbm, idx_hbm, gat_hbm, ones_hbm, sv, svi, iv, gv, ov):
    wid = lax.axis_index("s") * NC + lax.axis_index("c")

    @pl.when(wid < E)
    def _():
        e = wid
        pltpu.sync_copy(scores_hbm.at[pl.ds(e * S, S)], sv)

        def cast_body(i, _):
            svi[pl.ds(i * L, L)] = plsc.bitcast(sv[pl.ds(i * L, L)], jnp.int32)
            return 0
        lax.fori_loop(0, S // L, cast_body, 0, unroll=8)

        def count_ge(u):
            def body(i, acc):
                v = svi[pl.ds(i * L, L)]
                return acc + jnp.where(v >= u, 1, 0)
            accv = lax.fori_loop(0, S // L, body, jnp.zeros((L,), jnp.int32),
                                 unroll=8)
            return jnp.sum(accv)

        def bs_body(_, lohi):
            lo, hi = lohi
            mid = lo + (hi - lo) // 2
            big = count_ge(mid) >= K
            return (jnp.where(big, mid, lo), jnp.where(big, hi, mid))
        theta, _hi = lax.fori_loop(
            0, 31, bs_body, (jnp.int32(0), jnp.int32(0x7FFFFFFF)))
        n_gt = count_ge(theta + 1)
        r = K - n_gt

        lane = lax.iota(jnp.int32, L)

        def comp_body(i, carry):
            off, eqs = carry
            vi = svi[pl.ds(i * L, L)]
            vf = sv[pl.ds(i * L, L)]
            m_gt = vi > theta
            m_eq = vi == theta
            eq_i = jnp.where(m_eq, 1, 0)
            incl_eq = plsc.cumsum(eq_i)
            take = m_eq & ((eqs + (incl_eq - eq_i)) < r)
            m_sel = m_gt | take
            sel_i = jnp.where(m_sel, 1, 0)
            incl_sel = plsc.cumsum(sel_i)
            pos = off + incl_sel - 1
            plsc.store_scatter(iv, [pos], lane + i * L, mask=m_sel)
            plsc.store_scatter(gv, [pos], vf, mask=m_sel)
            ov[pl.ds(i * L, L)] = jnp.where(m_sel, 1.0, 0.0)
            return (off + jnp.max(incl_sel), eqs + jnp.max(incl_eq))
        lax.fori_loop(0, S // L, comp_body,
                      (jnp.int32(0), jnp.int32(0)), unroll=2)

        pltpu.sync_copy(iv, idx_hbm.at[pl.ds(e * K, K)])
        pltpu.sync_copy(gv, gat_hbm.at[pl.ds(e * K, K)])
        pltpu.sync_copy(ov, ones_hbm.at[pl.ds(e * S, S)])


def _sc_topk(scores_flat):
    mesh = plsc.VectorSubcoreMesh(core_axis_name="c", subcore_axis_name="s")
    return pl.kernel(
        _topk_body,
        out_type=(
            jax.ShapeDtypeStruct((EK,), jnp.int32),
            jax.ShapeDtypeStruct((EK,), jnp.float32),
            jax.ShapeDtypeStruct((E * S,), jnp.float32),
        ),
        mesh=mesh,
        scratch_types=[
            pltpu.VMEM((S,), jnp.float32),
            pltpu.VMEM((S,), jnp.int32),
            pltpu.VMEM((K,), jnp.int32),
            pltpu.VMEM((K,), jnp.float32),
            pltpu.VMEM((S,), jnp.float32),
        ],
        compiler_params=pltpu.CompilerParams(needs_layout_passes=False),
    )(scores_flat)


_GCH = 64


def _gather_body(xf_hbm, idx_hbm, xg_hbm, ivb, rows, sem):
    wid = lax.axis_index("s") * NC + lax.axis_index("c")
    rows_per = EK // NW
    for ch in range(rows_per // _GCH):
        base = wid * rows_per + ch * _GCH
        pltpu.sync_copy(idx_hbm.at[pl.ds(base, _GCH)], ivb)
        pltpu.async_copy(xf_hbm.at[ivb], rows, sem).wait()
        pltpu.sync_copy(rows, xg_hbm.at[pl.ds(base, _GCH)])


def _sc_gather(xf, idx_flat):
    mesh = plsc.VectorSubcoreMesh(core_axis_name="c", subcore_axis_name="s")
    return pl.kernel(
        _gather_body,
        out_type=jax.ShapeDtypeStruct((EK, D), jnp.float32),
        mesh=mesh,
        scratch_types=[
            pltpu.VMEM((_GCH,), jnp.int32),
            pltpu.VMEM((_GCH, D), jnp.float32),
            pltpu.SemaphoreType.DMA,
        ],
        compiler_params=pltpu.CompilerParams(needs_layout_passes=False),
    )(xf, idx_flat)


_HB = 512
_NHB = H // _HB


def _gelu_tanh(v):
    c = 0.7978845608028654
    return 0.5 * v * (1.0 + jnp.tanh(c * (v + 0.044715 * v * v * v)))


def _mlp_body(xg_ref, w1_ref, b1_ref, w2_ref, b2_ref, g_ref, out_ref):
    hb = pl.program_id(1)
    h = lax.dot_general(xg_ref[...], w1_ref[0], (((1,), (1,)), ((), ())),
                        preferred_element_type=jnp.float32,
                        precision=_PREC_MLP) + b1_ref[0]
    h = _gelu_tanh(h)
    part = lax.dot_general(h, w2_ref[0], (((1,), (1,)), ((), ())),
                           preferred_element_type=jnp.float32,
                           precision=_PREC_MLP)

    @pl.when(hb == 0)
    def _():
        out_ref[...] = part

    @pl.when(hb > 0)
    def _():
        out_ref[...] += part

    @pl.when(hb == _NHB - 1)
    def _():
        out_ref[...] = (out_ref[...] + b2_ref[0]) * g_ref[0, 0][:, None]


def _tc_mlp(xg, ew1, eb1, ew2, eb2, gating):
    grid = (E, _NHB)
    return pl.pallas_call(
        _mlp_body,
        grid=grid,
        in_specs=[
            pl.BlockSpec((K, D), lambda e, j: (e, 0)),
            pl.BlockSpec((1, _HB, D), lambda e, j: (e, j, 0)),
            pl.BlockSpec((1, 1, _HB), lambda e, j: (e * _NHB + j, 0, 0)),
            pl.BlockSpec((1, D, _HB), lambda e, j: (e, 0, j)),
            pl.BlockSpec((1, 1, D), lambda e, j: (e, 0, 0)),
            pl.BlockSpec((1, 1, K), lambda e, j: (e, 0, 0)),
        ],
        out_specs=pl.BlockSpec((K, D), lambda e, j: (e, 0)),
        out_shape=jax.ShapeDtypeStruct((EK, D), jnp.float32),
    )(xg, ew1, eb1.reshape(E * _NHB, 1, _HB), ew2,
      eb2.reshape(E, 1, D), gating.reshape(E, 1, K))


_TW = S // NW


def _combine_body(out_hbm, idx_hbm, y_hbm, acc, ie, fb, sem):
    wid = lax.axis_index("s") * NC + lax.axis_index("c")
    t0 = wid * _TW
    lane = lax.iota(jnp.int32, L)

    def zero_body(i, _):
        acc[pl.ds(i * L, L)] = jnp.zeros((L,), jnp.float32)
        return 0
    lax.fori_loop(0, _TW * D // L, zero_body, 0, unroll=8)

    def expert_body(e, _):
        pltpu.sync_copy(idx_hbm.at[pl.ds(e * K, K)], ie)

        def cnt_body(i, c):
            v = ie[pl.ds(i * L, L)]
            return (c[0] + jnp.where(v < t0, 1, 0),
                    c[1] + jnp.where(v < t0 + _TW, 1, 0))
        acc_lo, acc_hi = lax.fori_loop(
            0, K // L, cnt_body,
            (jnp.zeros((L,), jnp.int32), jnp.zeros((L,), jnp.int32)),
            unroll=8)
        lo = jnp.sum(acc_lo)
        hi = jnp.sum(acc_hi)

        def chunk_body(ch, _):
            base = lo + ch * L

            @pl.when(base < hi)
            def _():
                jv = jnp.minimum(base + lane, K - 1)
                tv = plsc.load_gather(ie, [jv]) - t0
                pltpu.async_copy(out_hbm.at[e * K + jv], fb, sem).wait()
                valid = hi - base
                for k in range(L):
                    @pl.when(k < valid)
                    def _(k=k, tv=tv):
                        tl = tv[k]

                        def add_body(i, _):
                            acc[pl.ds(tl * D + i * L, L)] = (
                                acc[pl.ds(tl * D + i * L, L)]
                                + fb[k, pl.ds(i * L, L)])
                            return 0
                        lax.fori_loop(0, D // L, add_body, 0, unroll=4)
            return 0
        lax.fori_loop(0, _TW // L, chunk_body, 0)
        return 0
    lax.fori_loop(0, E, expert_body, 0)

    pltpu.sync_copy(acc, y_hbm.at[pl.ds(t0 * D, _TW * D)])


def _sc_combine(out, idx_flat):
    mesh = plsc.VectorSubcoreMesh(core_axis_name="c", subcore_axis_name="s")
    y = pl.kernel(
        _combine_body,
        out_type=jax.ShapeDtypeStruct((S * D,), jnp.float32),
        mesh=mesh,
        scratch_types=[
            pltpu.VMEM((_TW * D,), jnp.float32),
            pltpu.VMEM((K,), jnp.int32),
            pltpu.VMEM((L, D), jnp.float32),
            pltpu.SemaphoreType.DMA,
        ],
        compiler_params=pltpu.CompilerParams(needs_layout_passes=False),
    )(out, idx_flat)
    return y


def kernel(x, gate_weight, cp_w1, cp_b1, cp_w2, cp_b2, ew1, eb1, ew2, eb2):
    B, Sx, Dx = x.shape
    xf = x.reshape(S, D)
    scoresT, cap = _tc_router(xf, gate_weight, cp_w1, cp_b1, cp_w2, cp_b2)
    idx_flat, gat_flat, ones_flat = _sc_topk(scoresT.reshape(-1))
    xg = _sc_gather(xf, idx_flat)
    out = _tc_mlp(xg, ew1, eb1, ew2, eb2, gat_flat.reshape(E, K))
    y = _sc_combine(out, idx_flat)
    x_out = y.reshape(B, Sx, Dx)
    ones_out = ones_flat.reshape(E, S).T.reshape(B, Sx, E)
    capacity_out = cap.reshape(B, Sx, E)
    return (x_out, ones_out, capacity_out)


# speedup vs baseline: 2.1841x; 2.1841x over previous
import jax
import jax.numpy as jnp
from jax import lax
from jax.experimental import pallas as pl
from jax.experimental.pallas import tpu as pltpu
from jax.experimental.pallas import tpu_sc as plsc

E = 8
D = 1024
H = 4096
S = 2048
K = 512
EK = E * K
L = 16
NC = 2
NS = 16
NW = NC * NS
DH = D // NC

_PREC_GATE = lax.Precision.DEFAULT
_PREC_CP = lax.Precision.DEFAULT
_PREC_MLP = lax.Precision.DEFAULT


def _router_body(x_ref, gw_ref, w1_ref, b1_ref, w2_ref, b2_ref,
                 scores_ref, cap_ref):
    xb = x_ref[...]
    lg = lax.dot_general(gw_ref[...], xb, (((1,), (1,)), ((), ())),
                         preferred_element_type=jnp.float32,
                         precision=_PREC_GATE)
    m = jnp.max(lg, axis=0, keepdims=True)
    p = jnp.exp(lg - m)
    scores_ref[...] = p / jnp.sum(p, axis=0, keepdims=True)
    h = lax.dot_general(xb, w1_ref[...], (((1,), (1,)), ((), ())),
                        preferred_element_type=jnp.float32,
                        precision=_PREC_CP) + b1_ref[...]
    h = h * (1.0 / (1.0 + jnp.exp(-h)))
    cap_ref[...] = lax.dot_general(h, w2_ref[...], (((1,), (1,)), ((), ())),
                                   precision=_PREC_CP,
                                   preferred_element_type=jnp.float32) + b2_ref[...]


def _tc_router(xf, gate_weight, cp_w1, cp_b1, cp_w2, cp_b2):
    BS = 512
    grid = (S // BS,)
    return pl.pallas_call(
        _router_body,
        grid=grid,
        in_specs=[
            pl.BlockSpec((BS, D), lambda i: (i, 0)),
            pl.BlockSpec((E, D), lambda i: (0, 0)),
            pl.BlockSpec((D, D), lambda i: (0, 0)),
            pl.BlockSpec((D,), lambda i: (0,)),
            pl.BlockSpec((E, D), lambda i: (0, 0)),
            pl.BlockSpec((E,), lambda i: (0,)),
        ],
        out_specs=[
            pl.BlockSpec((E, BS), lambda i: (0, i)),
            pl.BlockSpec((BS, E), lambda i: (i, 0)),
        ],
        out_shape=[
            jax.ShapeDtypeStruct((E, S), jnp.float32),
            jax.ShapeDtypeStruct((S, E), jnp.float32),
        ],
    )(xf, gate_weight, cp_w1, cp_b1, cp_w2, cp_b2)


def _topk_body(scores_hbm, idx_hbm, gat_hbm, ones_hbm, sv, svi, iv, gv, ov):
    wid = lax.axis_index("s") * NC + lax.axis_index("c")

    @pl.when(wid < E)
    def _():
        e = wid
        pltpu.sync_copy(scores_hbm.at[pl.ds(e * S, S)], sv)

        def cast_body(i, _):
            svi[pl.ds(i * L, L)] = plsc.bitcast(sv[pl.ds(i * L, L)], jnp.int32)
            return 0
        lax.fori_loop(0, S // L, cast_body, 0, unroll=8)

        def count_ge(u):
            def body(i, acc):
                v = svi[pl.ds(i * L, L)]
                return acc + jnp.where(v >= u, 1, 0)
            accv = lax.fori_loop(0, S // L, body, jnp.zeros((L,), jnp.int32),
                                 unroll=8)
            return jnp.sum(accv)

        def bs_body(_, lohi):
            lo, hi = lohi
            mid = lo + (hi - lo) // 2
            big = count_ge(mid) >= K
            return (jnp.where(big, mid, lo), jnp.where(big, hi, mid))
        theta, _hi = lax.fori_loop(
            0, 31, bs_body, (jnp.int32(0), jnp.int32(0x7FFFFFFF)))
        n_gt = count_ge(theta + 1)
        r = K - n_gt

        lane = lax.iota(jnp.int32, L)

        def comp_body(i, carry):
            off, eqs = carry
            vi = svi[pl.ds(i * L, L)]
            vf = sv[pl.ds(i * L, L)]
            m_gt = vi > theta
            m_eq = vi == theta
            eq_i = jnp.where(m_eq, 1, 0)
            incl_eq = plsc.cumsum(eq_i)
            take = m_eq & ((eqs + (incl_eq - eq_i)) < r)
            m_sel = m_gt | take
            sel_i = jnp.where(m_sel, 1, 0)
            incl_sel = plsc.cumsum(sel_i)
            pos = off + incl_sel - 1
            plsc.store_scatter(iv, [pos], lane + i * L, mask=m_sel)
            plsc.store_scatter(gv, [pos], vf, mask=m_sel)
            ov[pl.ds(i * L, L)] = jnp.where(m_sel, 1.0, 0.0)
            return (off + jnp.max(incl_sel), eqs + jnp.max(incl_eq))
        lax.fori_loop(0, S // L, comp_body,
                      (jnp.int32(0), jnp.int32(0)), unroll=2)

        pltpu.sync_copy(iv, idx_hbm.at[pl.ds(e * K, K)])
        pltpu.sync_copy(gv, gat_hbm.at[pl.ds(e * K, K)])
        pltpu.sync_copy(ov, ones_hbm.at[pl.ds(e * S, S)])


def _sc_topk(scores_flat):
    mesh = plsc.VectorSubcoreMesh(core_axis_name="c", subcore_axis_name="s")
    return pl.kernel(
        _topk_body,
        out_type=(
            jax.ShapeDtypeStruct((EK,), jnp.int32),
            jax.ShapeDtypeStruct((EK,), jnp.float32),
            jax.ShapeDtypeStruct((E * S,), jnp.float32),
        ),
        mesh=mesh,
        scratch_types=[
            pltpu.VMEM((S,), jnp.float32),
            pltpu.VMEM((S,), jnp.int32),
            pltpu.VMEM((K,), jnp.int32),
            pltpu.VMEM((K,), jnp.float32),
            pltpu.VMEM((S,), jnp.float32),
        ],
        compiler_params=pltpu.CompilerParams(needs_layout_passes=False),
    )(scores_flat)


_GCH = 64


def _gather_body(xf_hbm, idx_hbm, xg_hbm, ivb, rows, sem):
    wid = lax.axis_index("s") * NC + lax.axis_index("c")
    rows_per = EK // NW
    for ch in range(rows_per // _GCH):
        base = wid * rows_per + ch * _GCH
        pltpu.sync_copy(idx_hbm.at[pl.ds(base, _GCH)], ivb)
        pltpu.async_copy(xf_hbm.at[ivb], rows, sem).wait()
        pltpu.sync_copy(rows, xg_hbm.at[pl.ds(base, _GCH)])


def _sc_gather(xf, idx_flat):
    mesh = plsc.VectorSubcoreMesh(core_axis_name="c", subcore_axis_name="s")
    return pl.kernel(
        _gather_body,
        out_type=jax.ShapeDtypeStruct((EK, D), jnp.float32),
        mesh=mesh,
        scratch_types=[
            pltpu.VMEM((_GCH,), jnp.int32),
            pltpu.VMEM((_GCH, D), jnp.float32),
            pltpu.SemaphoreType.DMA,
        ],
        compiler_params=pltpu.CompilerParams(needs_layout_passes=False),
    )(xf, idx_flat)


_HB = 512
_NHB = H // _HB


def _gelu_tanh(v):
    c = 0.7978845608028654
    return 0.5 * v * (1.0 + jnp.tanh(c * (v + 0.044715 * v * v * v)))


def _mlp_body(xg_ref, w1_ref, b1_ref, w2_ref, b2_ref, g_ref, out_ref):
    hb = pl.program_id(1)
    h = lax.dot_general(xg_ref[...], w1_ref[0], (((1,), (1,)), ((), ())),
                        preferred_element_type=jnp.float32,
                        precision=_PREC_MLP) + b1_ref[0]
    h = _gelu_tanh(h)
    part = lax.dot_general(h, w2_ref[0], (((1,), (1,)), ((), ())),
                           preferred_element_type=jnp.float32,
                           precision=_PREC_MLP)

    @pl.when(hb == 0)
    def _():
        out_ref[...] = part

    @pl.when(hb > 0)
    def _():
        out_ref[...] += part

    @pl.when(hb == _NHB - 1)
    def _():
        out_ref[...] = (out_ref[...] + b2_ref[0]) * g_ref[0, 0][:, None]


def _tc_mlp(xg, ew1, eb1, ew2, eb2, gating):
    grid = (E, _NHB)
    return pl.pallas_call(
        _mlp_body,
        grid=grid,
        in_specs=[
            pl.BlockSpec((K, D), lambda e, j: (e, 0)),
            pl.BlockSpec((1, _HB, D), lambda e, j: (e, j, 0)),
            pl.BlockSpec((1, 1, _HB), lambda e, j: (e * _NHB + j, 0, 0)),
            pl.BlockSpec((1, D, _HB), lambda e, j: (e, 0, j)),
            pl.BlockSpec((1, 1, D), lambda e, j: (e, 0, 0)),
            pl.BlockSpec((1, 1, K), lambda e, j: (e, 0, 0)),
        ],
        out_specs=pl.BlockSpec((K, D), lambda e, j: (e, 0)),
        out_shape=jax.ShapeDtypeStruct((EK, D), jnp.float32),
    )(xg, ew1, eb1.reshape(E * _NHB, 1, _HB), ew2,
      eb2.reshape(E, 1, D), gating.reshape(E, 1, K))


_TW = S // NW


def _combine_body(out_hbm, idx_hbm, y_hbm, acc, ie, fb, sem):
    wid = lax.axis_index("s") * NC + lax.axis_index("c")
    t0 = wid * _TW
    lane = lax.iota(jnp.int32, L)

    def zero_body(i, _):
        acc[pl.ds(i * L, L)] = jnp.zeros((L,), jnp.float32)
        return 0
    lax.fori_loop(0, _TW * D // L, zero_body, 0, unroll=8)

    def expert_body(e, _):
        pltpu.sync_copy(idx_hbm.at[pl.ds(e * K, K)], ie)

        def cnt_body(i, c):
            v = ie[pl.ds(i * L, L)]
            return (c[0] + jnp.where(v < t0, 1, 0),
                    c[1] + jnp.where(v < t0 + _TW, 1, 0))
        acc_lo, acc_hi = lax.fori_loop(
            0, K // L, cnt_body,
            (jnp.zeros((L,), jnp.int32), jnp.zeros((L,), jnp.int32)),
            unroll=8)
        lo = jnp.sum(acc_lo)
        hi = jnp.sum(acc_hi)

        def chunk_body(ch, _):
            base = lo + ch * L

            @pl.when(base < hi)
            def _():
                jv = jnp.minimum(base + lane, K - 1)
                tv = plsc.load_gather(ie, [jv]) - t0
                pltpu.async_copy(out_hbm.at[e * K + jv], fb, sem).wait()
                valid = hi - base
                for k in range(L):
                    @pl.when(k < valid)
                    def _(k=k, tv=tv):
                        tl = tv[k]

                        def add_body(i, _):
                            acc[pl.ds(tl * D + i * L, L)] = (
                                acc[pl.ds(tl * D + i * L, L)]
                                + fb[k, pl.ds(i * L, L)])
                            return 0
                        lax.fori_loop(0, D // L, add_body, 0, unroll=4)
            return 0
        lax.fori_loop(0, _TW // L, chunk_body, 0)
        return 0
    lax.fori_loop(0, E, expert_body, 0)

    pltpu.sync_copy(acc, y_hbm.at[pl.ds(t0 * D, _TW * D)])


def _sc_combine(out, idx_flat):
    mesh = plsc.VectorSubcoreMesh(core_axis_name="c", subcore_axis_name="s")
    y = pl.kernel(
        _combine_body,
        out_type=jax.ShapeDtypeStruct((S * D,), jnp.float32),
        mesh=mesh,
        scratch_types=[
            pltpu.VMEM((_TW * D,), jnp.float32),
            pltpu.VMEM((K,), jnp.int32),
            pltpu.VMEM((L, D), jnp.float32),
            pltpu.SemaphoreType.DMA,
        ],
        compiler_params=pltpu.CompilerParams(needs_layout_passes=False),
    )(out, idx_flat)
    return y


def kernel(x, gate_weight, cp_w1, cp_b1, cp_w2, cp_b2, ew1, eb1, ew2, eb2):
    B, Sx, Dx = x.shape
    xf = x.reshape(S, D)
    scoresT, cap = _tc_router(xf, gate_weight, cp_w1, cp_b1, cp_w2, cp_b2)
    idx_flat, gat_flat, ones_flat = _sc_topk(scoresT.reshape(-1))
    xg = _sc_gather(xf, idx_flat)
    out = _tc_mlp(xg, ew1, eb1, ew2, eb2, gat_flat.reshape(E, K))
    y = _sc_combine(out, idx_flat)
    x_out = y.reshape(B, Sx, Dx)
    ones_out = ones_flat.reshape(E, S).T.reshape(B, Sx, E)
    capacity_out = cap.reshape(B, Sx, E)
    return (x_out, ones_out, capacity_out)


# speedup vs baseline: 2.2075x; 1.0107x over previous
import jax
import jax.numpy as jnp
from jax import lax
from jax.experimental import pallas as pl
from jax.experimental.pallas import tpu as pltpu
from jax.experimental.pallas import tpu_sc as plsc

E = 8
D = 1024
H = 4096
S = 2048
K = 512
EK = E * K
L = 16
NC = 2
NS = 16
NW = NC * NS
DH = D // NC

_PREC_GATE = lax.Precision.DEFAULT
_PREC_CP = lax.Precision.DEFAULT
_PREC_MLP = lax.Precision.DEFAULT


def _router_body(x_ref, gw_ref, w1_ref, b1_ref, w2_ref, b2_ref,
                 scores_ref, cap_ref):
    xb = x_ref[...]
    lg = lax.dot_general(gw_ref[...], xb, (((1,), (1,)), ((), ())),
                         preferred_element_type=jnp.float32,
                         precision=_PREC_GATE)
    m = jnp.max(lg, axis=0, keepdims=True)
    p = jnp.exp(lg - m)
    scores_ref[...] = p / jnp.sum(p, axis=0, keepdims=True)
    h = lax.dot_general(xb, w1_ref[...], (((1,), (1,)), ((), ())),
                        preferred_element_type=jnp.float32,
                        precision=_PREC_CP) + b1_ref[...]
    h = h * (1.0 / (1.0 + jnp.exp(-h)))
    cap_ref[...] = lax.dot_general(h, w2_ref[...], (((1,), (1,)), ((), ())),
                                   precision=_PREC_CP,
                                   preferred_element_type=jnp.float32) + b2_ref[...]


def _tc_router(xf, gate_weight, cp_w1, cp_b1, cp_w2, cp_b2):
    BS = 512
    grid = (S // BS,)
    return pl.pallas_call(
        _router_body,
        grid=grid,
        in_specs=[
            pl.BlockSpec((BS, D), lambda i: (i, 0)),
            pl.BlockSpec((E, D), lambda i: (0, 0)),
            pl.BlockSpec((D, D), lambda i: (0, 0)),
            pl.BlockSpec((D,), lambda i: (0,)),
            pl.BlockSpec((E, D), lambda i: (0, 0)),
            pl.BlockSpec((E,), lambda i: (0,)),
        ],
        out_specs=[
            pl.BlockSpec((E, BS), lambda i: (0, i)),
            pl.BlockSpec((BS, E), lambda i: (i, 0)),
        ],
        out_shape=[
            jax.ShapeDtypeStruct((E, S), jnp.float32),
            jax.ShapeDtypeStruct((S, E), jnp.float32),
        ],
    )(xf, gate_weight, cp_w1, cp_b1, cp_w2, cp_b2)


def _topk_body(scores_hbm, idx_hbm, gat_hbm, ones_hbm, sv, svi, iv, gv, ov):
    wid = lax.axis_index("s") * NC + lax.axis_index("c")

    @pl.when(wid < E)
    def _():
        e = wid
        pltpu.sync_copy(scores_hbm.at[pl.ds(e * S, S)], sv)

        def cast_body(i, _):
            svi[pl.ds(i * L, L)] = plsc.bitcast(sv[pl.ds(i * L, L)], jnp.int32)
            return 0
        lax.fori_loop(0, S // L, cast_body, 0, unroll=8)

        def count_ge(u):
            def body(i, acc):
                v = svi[pl.ds(i * L, L)]
                return acc + jnp.where(v >= u, 1, 0)
            accv = lax.fori_loop(0, S // L, body, jnp.zeros((L,), jnp.int32),
                                 unroll=8)
            return jnp.sum(accv)

        def bs_body(_, lohi):
            lo, hi = lohi
            mid = lo + (hi - lo) // 2
            big = count_ge(mid) >= K
            return (jnp.where(big, mid, lo), jnp.where(big, hi, mid))
        theta, _hi = lax.fori_loop(
            0, 31, bs_body, (jnp.int32(0), jnp.int32(0x7FFFFFFF)))
        n_gt = count_ge(theta + 1)
        r = K - n_gt

        lane = lax.iota(jnp.int32, L)

        def comp_body(i, carry):
            off, eqs = carry
            vi = svi[pl.ds(i * L, L)]
            vf = sv[pl.ds(i * L, L)]
            m_gt = vi > theta
            m_eq = vi == theta
            eq_i = jnp.where(m_eq, 1, 0)
            incl_eq = plsc.cumsum(eq_i)
            take = m_eq & ((eqs + (incl_eq - eq_i)) < r)
            m_sel = m_gt | take
            sel_i = jnp.where(m_sel, 1, 0)
            incl_sel = plsc.cumsum(sel_i)
            pos = off + incl_sel - 1
            plsc.store_scatter(iv, [pos], lane + i * L, mask=m_sel)
            plsc.store_scatter(gv, [pos], vf, mask=m_sel)
            ov[pl.ds(i * L, L)] = jnp.where(m_sel, 1.0, 0.0)
            return (off + jnp.max(incl_sel), eqs + jnp.max(incl_eq))
        lax.fori_loop(0, S // L, comp_body,
                      (jnp.int32(0), jnp.int32(0)), unroll=2)

        pltpu.sync_copy(iv, idx_hbm.at[pl.ds(e * K, K)])
        pltpu.sync_copy(gv, gat_hbm.at[pl.ds(e * K, K)])
        pltpu.sync_copy(ov, ones_hbm.at[pl.ds(e * S, S)])


def _sc_topk(scores_flat):
    mesh = plsc.VectorSubcoreMesh(core_axis_name="c", subcore_axis_name="s")
    return pl.kernel(
        _topk_body,
        out_type=(
            jax.ShapeDtypeStruct((EK,), jnp.int32),
            jax.ShapeDtypeStruct((EK,), jnp.float32),
            jax.ShapeDtypeStruct((E * S,), jnp.float32),
        ),
        mesh=mesh,
        scratch_types=[
            pltpu.VMEM((S,), jnp.float32),
            pltpu.VMEM((S,), jnp.int32),
            pltpu.VMEM((K,), jnp.int32),
            pltpu.VMEM((K,), jnp.float32),
            pltpu.VMEM((S,), jnp.float32),
        ],
        compiler_params=pltpu.CompilerParams(needs_layout_passes=False),
    )(scores_flat)


_GCH = 64


def _gather_body(xf_hbm, idx_hbm, xg_hbm, ivb, rows, sem):
    wid = lax.axis_index("s") * NC + lax.axis_index("c")
    rows_per = EK // NW
    for ch in range(rows_per // _GCH):
        base = wid * rows_per + ch * _GCH
        pltpu.sync_copy(idx_hbm.at[pl.ds(base, _GCH)], ivb)
        pltpu.async_copy(xf_hbm.at[ivb], rows, sem).wait()
        pltpu.sync_copy(rows, xg_hbm.at[pl.ds(base, _GCH)])


def _sc_gather(xf, idx_flat):
    mesh = plsc.VectorSubcoreMesh(core_axis_name="c", subcore_axis_name="s")
    return pl.kernel(
        _gather_body,
        out_type=jax.ShapeDtypeStruct((EK, D), jnp.float32),
        mesh=mesh,
        scratch_types=[
            pltpu.VMEM((_GCH,), jnp.int32),
            pltpu.VMEM((_GCH, D), jnp.float32),
            pltpu.SemaphoreType.DMA,
        ],
        compiler_params=pltpu.CompilerParams(needs_layout_passes=False),
    )(xf, idx_flat)


_HB = 512
_NHB = H // _HB


def _gelu_tanh(v):
    c = 0.7978845608028654
    return 0.5 * v * (1.0 + jnp.tanh(c * (v + 0.044715 * v * v * v)))


def _mlp_body(xg_ref, w1_ref, b1_ref, w2_ref, b2_ref, g_ref, out_ref):
    hb = pl.program_id(1)
    h = lax.dot_general(xg_ref[...], w1_ref[0], (((1,), (1,)), ((), ())),
                        preferred_element_type=jnp.float32,
                        precision=_PREC_MLP) + b1_ref[0]
    h = _gelu_tanh(h)
    part = lax.dot_general(h, w2_ref[0], (((1,), (1,)), ((), ())),
                           preferred_element_type=jnp.float32,
                           precision=_PREC_MLP)

    @pl.when(hb == 0)
    def _():
        out_ref[...] = part

    @pl.when(hb > 0)
    def _():
        out_ref[...] += part

    @pl.when(hb == _NHB - 1)
    def _():
        out_ref[...] = (out_ref[...] + b2_ref[0]) * g_ref[0, 0][:, None]


def _tc_mlp(xg, ew1, eb1, ew2, eb2, gating):
    grid = (E, _NHB)
    return pl.pallas_call(
        _mlp_body,
        grid=grid,
        in_specs=[
            pl.BlockSpec((K, D), lambda e, j: (e, 0)),
            pl.BlockSpec((1, _HB, D), lambda e, j: (e, j, 0)),
            pl.BlockSpec((1, 1, _HB), lambda e, j: (e * _NHB + j, 0, 0)),
            pl.BlockSpec((1, D, _HB), lambda e, j: (e, 0, j)),
            pl.BlockSpec((1, 1, D), lambda e, j: (e, 0, 0)),
            pl.BlockSpec((1, 1, K), lambda e, j: (e, 0, 0)),
        ],
        out_specs=pl.BlockSpec((K, D), lambda e, j: (e, 0)),
        out_shape=jax.ShapeDtypeStruct((EK, D), jnp.float32),
    )(xg, ew1, eb1.reshape(E * _NHB, 1, _HB), ew2,
      eb2.reshape(E, 1, D), gating.reshape(E, 1, K))


_TW = S // NW


def _combine_body(out_hbm, idx_hbm, y_hbm, acc, idxall, fb, tvb, sem0, sem1):
    wid = lax.axis_index("s") * NC + lax.axis_index("c")
    t0 = wid * _TW
    lane = lax.iota(jnp.int32, L)
    zero16 = jnp.zeros((L,), jnp.int32)

    pltpu.sync_copy(idx_hbm, idxall)

    def zero_body(i, _):
        acc[pl.ds(i * L, L)] = jnp.zeros((L,), jnp.float32)
        return 0
    lax.fori_loop(0, _TW * D // L, zero_body, 0, unroll=8)

    los, his = [], []
    for e in range(E):
        def cnt_body(i, c, e=e):
            v = idxall[pl.ds(e * K + i * L, L)]
            return (c[0] + jnp.where(v < t0, 1, 0),
                    c[1] + jnp.where(v < t0 + _TW, 1, 0))
        alo, ahi = lax.fori_loop(0, K // L, cnt_body, (zero16, zero16),
                                 unroll=8)
        los.append(jnp.sum(alo))
        his.append(jnp.sum(ahi))

    sems = (sem0, sem1)
    slots = [(e, ch) for e in range(E) for ch in range(_TW // L)]

    def slot_state(i):
        e, ch = slots[i]
        base = los[e] + ch * L
        return e, base, his[e]

    def start_slot(i):
        e, base, hi = slot_state(i)
        p = i % 2

        @pl.when(base < hi)
        def _():
            jv = jnp.minimum(base + lane, K - 1)
            tvb[p, pl.ds(0, L)] = plsc.load_gather(idxall, [e * K + jv]) - t0
            pltpu.async_copy(out_hbm.at[e * K + jv], fb.at[p], sems[p])

    def drain_slot(i):
        e, base, hi = slot_state(i)
        p = i % 2

        @pl.when(base < hi)
        def _():
            pltpu.make_async_copy(out_hbm.at[pl.ds(0, L)], fb.at[p],
                                  sems[p]).wait()
            vlen = jnp.minimum(hi - base, L)

            def k_body(k, _):
                tl = tvb[p, pl.ds(k, L)][0]

                def add_body(ii, _):
                    off = tl * D + ii * L
                    acc[pl.ds(off, L)] = (acc[pl.ds(off, L)]
                                          + fb[p, k, pl.ds(ii * L, L)])
                    return 0
                lax.fori_loop(0, D // L, add_body, 0, unroll=4)
                return 0
            lax.fori_loop(0, vlen, k_body, 0)

    for i in range(len(slots)):
        start_slot(i)
        if i > 0:
            drain_slot(i - 1)
    drain_slot(len(slots) - 1)

    pltpu.sync_copy(acc, y_hbm.at[pl.ds(t0 * D, _TW * D)])


def _sc_combine(out, idx_flat):
    mesh = plsc.VectorSubcoreMesh(core_axis_name="c", subcore_axis_name="s")
    y = pl.kernel(
        _combine_body,
        out_type=jax.ShapeDtypeStruct((S * D,), jnp.float32),
        mesh=mesh,
        scratch_types=[
            pltpu.VMEM((_TW * D,), jnp.float32),
            pltpu.VMEM((EK,), jnp.int32),
            pltpu.VMEM((2, L, D), jnp.float32),
            pltpu.VMEM((2, 2 * L), jnp.int32),
            pltpu.SemaphoreType.DMA,
            pltpu.SemaphoreType.DMA,
        ],
        compiler_params=pltpu.CompilerParams(needs_layout_passes=False),
    )(out, idx_flat)
    return y


def kernel(x, gate_weight, cp_w1, cp_b1, cp_w2, cp_b2, ew1, eb1, ew2, eb2):
    B, Sx, Dx = x.shape
    xf = x.reshape(S, D)
    scoresT, cap = _tc_router(xf, gate_weight, cp_w1, cp_b1, cp_w2, cp_b2)
    idx_flat, gat_flat, ones_flat = _sc_topk(scoresT.reshape(-1))
    xg = _sc_gather(xf, idx_flat)
    out = _tc_mlp(xg, ew1, eb1, ew2, eb2, gat_flat.reshape(E, K))
    y = _sc_combine(out, idx_flat)
    x_out = y.reshape(B, Sx, Dx)
    ones_out = ones_flat.reshape(E, S).T.reshape(B, Sx, E)
    capacity_out = cap.reshape(B, Sx, E)
    return (x_out, ones_out, capacity_out)


# speedup vs baseline: 2.5434x; 1.1522x over previous
import jax
import jax.numpy as jnp
from jax import lax
from jax.experimental import pallas as pl
from jax.experimental.pallas import tpu as pltpu
from jax.experimental.pallas import tpu_sc as plsc

E = 8
D = 1024
H = 4096
S = 2048
K = 512
EK = E * K
L = 16
NC = 2
NS = 16
NW = NC * NS
DH = D // NC

_PREC_GATE = lax.Precision.DEFAULT
_PREC_CP = lax.Precision.DEFAULT
_PREC_MLP = lax.Precision.DEFAULT


def _router_body(x_ref, gw_ref, w1_ref, b1_ref, w2_ref, b2_ref,
                 scores_ref, cap_ref):
    xb = x_ref[...]
    lg = lax.dot_general(gw_ref[...], xb, (((1,), (1,)), ((), ())),
                         preferred_element_type=jnp.float32,
                         precision=_PREC_GATE)
    m = jnp.max(lg, axis=0, keepdims=True)
    p = jnp.exp(lg - m)
    scores_ref[...] = p / jnp.sum(p, axis=0, keepdims=True)
    h = lax.dot_general(xb, w1_ref[...], (((1,), (1,)), ((), ())),
                        preferred_element_type=jnp.float32,
                        precision=_PREC_CP) + b1_ref[...]
    h = h * (1.0 / (1.0 + jnp.exp(-h)))
    cap_ref[...] = lax.dot_general(h, w2_ref[...], (((1,), (1,)), ((), ())),
                                   precision=_PREC_CP,
                                   preferred_element_type=jnp.float32) + b2_ref[...]


def _tc_router(xf, gate_weight, cp_w1, cp_b1, cp_w2, cp_b2):
    BS = 512
    grid = (S // BS,)
    return pl.pallas_call(
        _router_body,
        grid=grid,
        in_specs=[
            pl.BlockSpec((BS, D), lambda i: (i, 0)),
            pl.BlockSpec((E, D), lambda i: (0, 0)),
            pl.BlockSpec((D, D), lambda i: (0, 0)),
            pl.BlockSpec((D,), lambda i: (0,)),
            pl.BlockSpec((E, D), lambda i: (0, 0)),
            pl.BlockSpec((E,), lambda i: (0,)),
        ],
        out_specs=[
            pl.BlockSpec((E, BS), lambda i: (0, i)),
            pl.BlockSpec((BS, E), lambda i: (i, 0)),
        ],
        out_shape=[
            jax.ShapeDtypeStruct((E, S), jnp.float32),
            jax.ShapeDtypeStruct((S, E), jnp.float32),
        ],
    )(xf, gate_weight, cp_w1, cp_b1, cp_w2, cp_b2)


def _topk_body(scores_hbm, idx_hbm, gat_hbm, ones_hbm, sv, svi, iv, gv, ov):
    wid = lax.axis_index("s") * NC + lax.axis_index("c")

    @pl.when(wid < E)
    def _():
        e = wid
        pltpu.sync_copy(scores_hbm.at[pl.ds(e * S, S)], sv)

        def cast_body(i, _):
            svi[pl.ds(i * L, L)] = plsc.bitcast(sv[pl.ds(i * L, L)], jnp.int32)
            return 0
        lax.fori_loop(0, S // L, cast_body, 0, unroll=8)

        def count_ge(u):
            def body(i, acc):
                v = svi[pl.ds(i * L, L)]
                return acc + jnp.where(v >= u, 1, 0)
            accv = lax.fori_loop(0, S // L, body, jnp.zeros((L,), jnp.int32),
                                 unroll=8)
            return jnp.sum(accv)

        def bs_body(_, lohi):
            lo, hi = lohi
            mid = lo + (hi - lo) // 2
            big = count_ge(mid) >= K
            return (jnp.where(big, mid, lo), jnp.where(big, hi, mid))
        theta, _hi = lax.fori_loop(
            0, 31, bs_body, (jnp.int32(0), jnp.int32(0x7FFFFFFF)))
        n_gt = count_ge(theta + 1)
        r = K - n_gt

        lane = lax.iota(jnp.int32, L)

        def comp_body(i, carry):
            off, eqs = carry
            vi = svi[pl.ds(i * L, L)]
            vf = sv[pl.ds(i * L, L)]
            m_gt = vi > theta
            m_eq = vi == theta
            eq_i = jnp.where(m_eq, 1, 0)
            incl_eq = plsc.cumsum(eq_i)
            take = m_eq & ((eqs + (incl_eq - eq_i)) < r)
            m_sel = m_gt | take
            sel_i = jnp.where(m_sel, 1, 0)
            incl_sel = plsc.cumsum(sel_i)
            pos = off + incl_sel - 1
            plsc.store_scatter(iv, [pos], lane + i * L, mask=m_sel)
            plsc.store_scatter(gv, [pos], vf, mask=m_sel)
            ov[pl.ds(i * L, L)] = jnp.where(m_sel, 1.0, 0.0)
            return (off + jnp.max(incl_sel), eqs + jnp.max(incl_eq))
        lax.fori_loop(0, S // L, comp_body,
                      (jnp.int32(0), jnp.int32(0)), unroll=2)

        pltpu.sync_copy(iv, idx_hbm.at[pl.ds(e * K, K)])
        pltpu.sync_copy(gv, gat_hbm.at[pl.ds(e * K, K)])
        pltpu.sync_copy(ov, ones_hbm.at[pl.ds(e * S, S)])


def _sc_topk(scores_flat):
    mesh = plsc.VectorSubcoreMesh(core_axis_name="c", subcore_axis_name="s")
    return pl.kernel(
        _topk_body,
        out_type=(
            jax.ShapeDtypeStruct((EK,), jnp.int32),
            jax.ShapeDtypeStruct((EK,), jnp.float32),
            jax.ShapeDtypeStruct((E * S,), jnp.float32),
        ),
        mesh=mesh,
        scratch_types=[
            pltpu.VMEM((S,), jnp.float32),
            pltpu.VMEM((S,), jnp.int32),
            pltpu.VMEM((K,), jnp.int32),
            pltpu.VMEM((K,), jnp.float32),
            pltpu.VMEM((S,), jnp.float32),
        ],
        compiler_params=pltpu.CompilerParams(needs_layout_passes=False),
    )(scores_flat)


_GCH = 64


def _gather_body(xf_hbm, idx_hbm, xg_hbm, ivb, rows, sem):
    wid = lax.axis_index("s") * NC + lax.axis_index("c")
    rows_per = EK // NW
    for ch in range(rows_per // _GCH):
        base = wid * rows_per + ch * _GCH
        pltpu.sync_copy(idx_hbm.at[pl.ds(base, _GCH)], ivb)
        pltpu.async_copy(xf_hbm.at[ivb], rows, sem).wait()
        pltpu.sync_copy(rows, xg_hbm.at[pl.ds(base, _GCH)])


def _sc_gather(xf, idx_flat):
    mesh = plsc.VectorSubcoreMesh(core_axis_name="c", subcore_axis_name="s")
    return pl.kernel(
        _gather_body,
        out_type=jax.ShapeDtypeStruct((EK, D), jnp.float32),
        mesh=mesh,
        scratch_types=[
            pltpu.VMEM((_GCH,), jnp.int32),
            pltpu.VMEM((_GCH, D), jnp.float32),
            pltpu.SemaphoreType.DMA,
        ],
        compiler_params=pltpu.CompilerParams(needs_layout_passes=False),
    )(xf, idx_flat)


_HB = 512
_NHB = H // _HB


def _gelu_tanh(v):
    c = 0.7978845608028654
    return 0.5 * v * (1.0 + jnp.tanh(c * (v + 0.044715 * v * v * v)))


def _mlp_body(xg_ref, w1_ref, b1_ref, w2_ref, b2_ref, g_ref, out_ref):
    hb = pl.program_id(1)
    h = lax.dot_general(xg_ref[...], w1_ref[0], (((1,), (1,)), ((), ())),
                        preferred_element_type=jnp.float32,
                        precision=_PREC_MLP) + b1_ref[0]
    h = _gelu_tanh(h)
    part = lax.dot_general(h, w2_ref[0], (((1,), (1,)), ((), ())),
                           preferred_element_type=jnp.float32,
                           precision=_PREC_MLP)

    @pl.when(hb == 0)
    def _():
        out_ref[...] = part

    @pl.when(hb > 0)
    def _():
        out_ref[...] += part

    @pl.when(hb == _NHB - 1)
    def _():
        out_ref[...] = (out_ref[...] + b2_ref[0]) * g_ref[0, 0][:, None]


def _tc_mlp(xg, ew1, eb1, ew2, eb2, gating):
    grid = (E, _NHB)
    return pl.pallas_call(
        _mlp_body,
        grid=grid,
        in_specs=[
            pl.BlockSpec((K, D), lambda e, j: (e, 0)),
            pl.BlockSpec((1, _HB, D), lambda e, j: (e, j, 0)),
            pl.BlockSpec((1, 1, _HB), lambda e, j: (e * _NHB + j, 0, 0)),
            pl.BlockSpec((1, D, _HB), lambda e, j: (e, 0, j)),
            pl.BlockSpec((1, 1, D), lambda e, j: (e, 0, 0)),
            pl.BlockSpec((1, 1, K), lambda e, j: (e, 0, 0)),
        ],
        out_specs=pl.BlockSpec((K, D), lambda e, j: (e, 0)),
        out_shape=jax.ShapeDtypeStruct((EK, D), jnp.float32),
    )(xg, ew1, eb1.reshape(E * _NHB, 1, _HB), ew2,
      eb2.reshape(E, 1, D), gating.reshape(E, 1, K))


_TW = S // NW


def _combine_body(out_hbm, idx_hbm, y_hbm, acc, idxall, fb, tvb, sem0, sem1):
    wid = lax.axis_index("s") * NC + lax.axis_index("c")
    t0 = wid * _TW
    lane = lax.iota(jnp.int32, L)
    zero16 = jnp.zeros((L,), jnp.int32)

    pltpu.sync_copy(idx_hbm, idxall)

    @plsc.parallel_loop(0, _TW * D, step=L, unroll=8)
    def _(i):
        acc[pl.ds(i, L)] = jnp.zeros((L,), jnp.float32)

    los, his = [], []
    for e in range(E):
        def cnt_body(i, c, e=e):
            v = idxall[pl.ds(e * K + i * L, L)]
            return (c[0] + jnp.where(v < t0, 1, 0),
                    c[1] + jnp.where(v < t0 + _TW, 1, 0))
        alo, ahi = lax.fori_loop(0, K // L, cnt_body, (zero16, zero16),
                                 unroll=8)
        los.append(jnp.sum(alo))
        his.append(jnp.sum(ahi))

    sems = (sem0, sem1)
    slots = [(e, ch) for e in range(E) for ch in range(_TW // L)]

    def slot_state(i):
        e, ch = slots[i]
        base = los[e] + ch * L
        return e, base, his[e]

    def start_slot(i):
        e, base, hi = slot_state(i)
        p = i % 2

        @pl.when(base < hi)
        def _():
            jv = jnp.minimum(base + lane, K - 1)
            tvb[p, pl.ds(0, L)] = plsc.load_gather(idxall, [e * K + jv]) - t0
            pltpu.async_copy(out_hbm.at[e * K + jv], fb.at[p], sems[p])

    def drain_slot(i):
        e, base, hi = slot_state(i)
        p = i % 2

        @pl.when(base < hi)
        def _():
            pltpu.make_async_copy(out_hbm.at[pl.ds(0, L)], fb.at[p],
                                  sems[p]).wait()
            vlen = jnp.minimum(hi - base, L)

            @plsc.parallel_loop(0, vlen)
            def _(k):
                tl = tvb[p, pl.ds(k, L)][0]
                rowbase = tl * D

                @plsc.parallel_loop(0, D, step=L, unroll=8)
                def _(ii):
                    acc[pl.ds(rowbase + ii, L)] = (
                        acc[pl.ds(rowbase + ii, L)]
                        + fb[p, k, pl.ds(ii, L)])

    for i in range(len(slots)):
        start_slot(i)
        if i > 0:
            drain_slot(i - 1)
    drain_slot(len(slots) - 1)

    pltpu.sync_copy(acc, y_hbm.at[pl.ds(t0 * D, _TW * D)])


def _sc_combine(out, idx_flat):
    mesh = plsc.VectorSubcoreMesh(core_axis_name="c", subcore_axis_name="s")
    y = pl.kernel(
        _combine_body,
        out_type=jax.ShapeDtypeStruct((S * D,), jnp.float32),
        mesh=mesh,
        scratch_types=[
            pltpu.VMEM((_TW * D,), jnp.float32),
            pltpu.VMEM((EK,), jnp.int32),
            pltpu.VMEM((2, L, D), jnp.float32),
            pltpu.VMEM((2, 2 * L), jnp.int32),
            pltpu.SemaphoreType.DMA,
            pltpu.SemaphoreType.DMA,
        ],
        compiler_params=pltpu.CompilerParams(needs_layout_passes=False),
    )(out, idx_flat)
    return y


def kernel(x, gate_weight, cp_w1, cp_b1, cp_w2, cp_b2, ew1, eb1, ew2, eb2):
    B, Sx, Dx = x.shape
    xf = x.reshape(S, D)
    scoresT, cap = _tc_router(xf, gate_weight, cp_w1, cp_b1, cp_w2, cp_b2)
    idx_flat, gat_flat, ones_flat = _sc_topk(scoresT.reshape(-1))
    xg = _sc_gather(xf, idx_flat)
    out = _tc_mlp(xg, ew1, eb1, ew2, eb2, gat_flat.reshape(E, K))
    y = _sc_combine(out, idx_flat)
    x_out = y.reshape(B, Sx, Dx)
    ones_out = ones_flat.reshape(E, S).T.reshape(B, Sx, E)
    capacity_out = cap.reshape(B, Sx, E)
    return (x_out, ones_out, capacity_out)
